# Initial kernel scaffold; baseline (speedup 1.0000x reference)
import jax
import jax.numpy as jnp
from jax.experimental import pallas as pl


def kernel(user, pos_item, neg_item, user_table, item_table):
    raise NotImplementedError("write your pallas kernel here")


# speedup vs baseline: 1.2403x; 1.2403x over previous
import functools
import jax
import jax.numpy as jnp
from jax import lax
from jax.experimental import pallas as pl
from jax.experimental.pallas import tpu as pltpu
from jax.experimental.pallas import tpu_sc as plsc

B = 16384
D = 128
L = 16
NW = 32
B_PER_W = B // NW
CH = 128
N_CHUNK = B_PER_W // CH


def _bpr_body(user_hbm, pos_hbm, neg_hbm, utab_hbm, itab_hbm, out_hbm,
              idx_u, idx_i, idx_j, rows_u, rows_i, rows_j, out_v,
              sem_u, sem_i, sem_j):
    wid = lax.axis_index("s") * 2 + lax.axis_index("c")
    base = wid * B_PER_W

    def chunk_body(ch, _):
        off = base + ch * CH
        pltpu.sync_copy(user_hbm.at[pl.ds(off, CH)], idx_u)
        pltpu.sync_copy(pos_hbm.at[pl.ds(off, CH)], idx_i)
        pltpu.sync_copy(neg_hbm.at[pl.ds(off, CH)], idx_j)
        cu = pltpu.async_copy(utab_hbm.at[idx_u], rows_u, sem_u)
        ci = pltpu.async_copy(itab_hbm.at[idx_i], rows_i, sem_i)
        cj = pltpu.async_copy(itab_hbm.at[idx_j], rows_j, sem_j)
        cu.wait()
        ci.wait()
        cj.wait()

        lane = lax.iota(jnp.int32, L)

        def group_body(g, _):
            res = jnp.zeros((L,), jnp.float32)
            for r16 in range(L):
                r = g * L + r16
                acc = jnp.zeros((L,), jnp.float32)
                for c in range(D // L):
                    uv = rows_u[r, pl.ds(c * L, L)]
                    iv = rows_i[r, pl.ds(c * L, L)]
                    jv = rows_j[r, pl.ds(c * L, L)]
                    acc = acc + uv * (iv - jv)
                s = jnp.sum(acc)
                res = jnp.where(lane == r16, s, res)
            out_v[pl.ds(g * L, L)] = res
            return 0

        lax.fori_loop(0, CH // L, group_body, 0)
        pltpu.sync_copy(out_v, out_hbm.at[pl.ds(off, CH)])
        return 0

    lax.fori_loop(0, N_CHUNK, chunk_body, 0)


@jax.jit
def _bpr(user, pos_item, neg_item, user_table, item_table):
    mesh = plsc.VectorSubcoreMesh(core_axis_name="c", subcore_axis_name="s")
    f = functools.partial(
        pl.kernel,
        mesh=mesh,
        compiler_params=pltpu.CompilerParams(needs_layout_passes=False),
        out_type=jax.ShapeDtypeStruct((B,), jnp.float32),
        scratch_types=[
            pltpu.VMEM((CH,), jnp.int32),
            pltpu.VMEM((CH,), jnp.int32),
            pltpu.VMEM((CH,), jnp.int32),
            pltpu.VMEM((CH, D), jnp.float32),
            pltpu.VMEM((CH, D), jnp.float32),
            pltpu.VMEM((CH, D), jnp.float32),
            pltpu.VMEM((CH,), jnp.float32),
            pltpu.SemaphoreType.DMA,
            pltpu.SemaphoreType.DMA,
            pltpu.SemaphoreType.DMA,
        ],
    )(_bpr_body)
    return f(user, pos_item, neg_item, user_table, item_table)


def kernel(user, pos_item, neg_item, user_table, item_table):
    return _bpr(user, pos_item, neg_item, user_table, item_table)
